# baseline (speedup 1.0000x reference)
import functools

import jax
import jax.numpy as jnp
from jax import lax
from jax.experimental import pallas as pl
from jax.experimental.pallas import tpu as pltpu
from jax.experimental.pallas import tpu_sc as plsc

N = 10000
F = 128
E = 320000
B = 2
NSUB = 16
NCORE = 2
CHUNK = 128
NCHUNK = 160
NK2 = NCHUNK // 2
EPS = NCHUNK * CHUNK
E_PAD = EPS * NSUB
GARB = N
ACC_ROWS = N + 8
RPN = N // NSUB
NSTG = 5
ZS_ROWS = 15 * RPN + NSTG * CHUNK
RPS_A = 632
RPS_LAST = N - 15 * RPS_A

_MESH = plsc.VectorSubcoreMesh(core_axis_name="c", subcore_axis_name="s")


def _rowcopy(sub, copy_fn):
    @pl.when(sub < NSUB - 1)
    def _():
        copy_fn(sub * RPS_A, RPS_A)

    @pl.when(sub == NSUB - 1)
    def _():
        copy_fn((NSUB - 1) * RPS_A, RPS_LAST)


def _agg128_kernel():
    @functools.partial(
        pl.kernel,
        out_type=jax.ShapeDtypeStruct((2 * NCORE * N, 64), jnp.float32),
        mesh=_MESH,
        compiler_params=pltpu.CompilerParams(use_tc_tiling_on_sc=False),
        scratch_types=[
            pltpu.VMEM((2, CHUNK), jnp.int32),
            pltpu.VMEM((2, CHUNK), jnp.int32),
            pltpu.VMEM((2, CHUNK), jnp.int32),
            pltpu.VMEM((2, CHUNK), jnp.int32),
            pltpu.VMEM((CHUNK,), jnp.int32),
            pltpu.VMEM((CHUNK,), jnp.int32),
            pltpu.VMEM((CHUNK, 64), jnp.float32),
            pltpu.VMEM((CHUNK, 64), jnp.float32),
            pltpu.VMEM_SHARED((ACC_ROWS, 64), jnp.float32),
            pltpu.VMEM_SHARED((ZS_ROWS, 64), jnp.float32),
            pltpu.SemaphoreType.DMA,
            pltpu.SemaphoreType.DMA,
            pltpu.SemaphoreType.DMA,
            pltpu.SemaphoreType.DMA,
        ],
    )
    def agg(z64, src_hbm, dst_hbm, zero_hbm, out64,
            sb, db, gb, wb, stg, og, rows0, rows1, acc, z_s,
            semi0, semi1, semg0, semg1):
        core = lax.axis_index("c")
        sub = lax.axis_index("s")
        base = sub * EPS
        rows = (rows0, rows1)
        semi = (semi0, semi1)
        semg = (semg0, semg1)
        iota = lax.iota(jnp.int32, 16)

        def idx_start(c, p):
            off = base + c * CHUNK
            pltpu.async_copy(src_hbm.at[pl.ds(off, CHUNK)], sb.at[p], semi[p])
            pltpu.async_copy(dst_hbm.at[pl.ds(off, CHUNK)], db.at[p], semi[p])

        def idx_wait(p):
            pltpu.make_async_copy(src_hbm.at[pl.ds(0, CHUNK)], sb.at[p], semi[p]).wait()
            pltpu.make_async_copy(dst_hbm.at[pl.ds(0, CHUNK)], db.at[p], semi[p]).wait()

        def make_ops(hoff):
            def transform(p):
                for j in range(CHUNK // 16):
                    sl = pl.ds(j * 16, 16)
                    s16 = sb[p, sl]
                    d16 = db[p, sl]
                    gb[p, sl] = 2 * s16 + hoff if p == 0 else s16
                    wb[p, sl] = jnp.where(s16 == d16, GARB, d16)

            def gather_start(p):
                tbl = z64 if p == 0 else z_s
                pltpu.async_copy(tbl.at[gb.at[p]], rows[p], semg[p])

            def gather_wait(p):
                tbl = z64 if p == 0 else z_s
                pltpu.make_async_copy(tbl.at[gb.at[p]], rows[p], semg[p]).wait()

            def scatter(p):
                pltpu.sync_copy(rows[p], acc.at[wb.at[p]], add=True)

            return transform, gather_start, gather_wait, scatter

        for h in range(2):
            hoff = 2 * core * N + h
            transform, gather_start, gather_wait, scatter = make_ops(hoff)
            _rowcopy(sub, lambda r0, nr: pltpu.sync_copy(
                zero_hbm.at[pl.ds(0, nr)], acc.at[pl.ds(r0, nr)]))
            for k in range(NSTG):
                r0 = sub * RPN + k * CHUNK
                for j in range(CHUNK // 16):
                    m = jnp.minimum(r0 + j * 16 + iota, N - 1)
                    stg[pl.ds(j * 16, 16)] = 2 * m + hoff
                pltpu.sync_copy(z64.at[stg], rows0)
                pltpu.sync_copy(rows0, z_s.at[pl.ds(r0, CHUNK)])
            plsc.subcore_barrier()

            idx_start(0, 0)
            idx_start(1, 1)
            idx_wait(0)
            transform(0)
            gather_start(0)
            idx_start(2, 0)

            def body(k2, carry):
                idx_wait(1)
                transform(1)
                gather_start(1)

                @pl.when(k2 < NK2 - 1)
                def _():
                    idx_start(2 * k2 + 3, 1)

                gather_wait(0)
                scatter(0)

                @pl.when(k2 < NK2 - 1)
                def _():
                    idx_wait(0)
                    transform(0)
                    gather_start(0)

                @pl.when(k2 < NK2 - 2)
                def _():
                    idx_start(2 * k2 + 4, 0)

                gather_wait(1)
                scatter(1)
                return carry

            lax.fori_loop(0, NK2, body, 0)
            plsc.subcore_barrier()

            for k in range(NSTG):
                r0 = sub * RPN + k * CHUNK
                for j in range(CHUNK // 16):
                    m = jnp.minimum(r0 + j * 16 + iota, N - 1)
                    stg[pl.ds(j * 16, 16)] = m
                    og[pl.ds(j * 16, 16)] = 2 * m + hoff
                pltpu.sync_copy(acc.at[stg], rows0)
                pltpu.sync_copy(rows0, out64.at[og])
            plsc.subcore_barrier()

    return agg


_agg128 = _agg128_kernel()


@functools.partial(
    pl.kernel,
    out_type=jax.ShapeDtypeStruct((NCORE * N, 16), jnp.float32),
    mesh=_MESH,
    compiler_params=pltpu.CompilerParams(use_tc_tiling_on_sc=False),
    scratch_types=[
        pltpu.VMEM((2, CHUNK), jnp.int32),
        pltpu.VMEM((2, CHUNK), jnp.int32),
        pltpu.VMEM((2, CHUNK), jnp.int32),
        pltpu.VMEM((2, CHUNK), jnp.int32),
        pltpu.VMEM((CHUNK, 16), jnp.float32),
        pltpu.VMEM((CHUNK, 16), jnp.float32),
        pltpu.VMEM_SHARED((ACC_ROWS, 16), jnp.float32),
        pltpu.VMEM_SHARED((N, 16), jnp.float32),
        pltpu.SemaphoreType.DMA,
        pltpu.SemaphoreType.DMA,
        pltpu.SemaphoreType.DMA,
        pltpu.SemaphoreType.DMA,
    ],
)
def _agg16(z_hbm, src_hbm, dst_hbm, out_hbm,
           sb, db, gb, wb, rows0, rows1, acc, z_s,
           semi0, semi1, semg0, semg1):
    core = lax.axis_index("c")
    sub = lax.axis_index("s")
    _rowcopy(sub, lambda r0, nr: pltpu.sync_copy(
        z_hbm.at[pl.ds(core * N + r0, nr)], acc.at[pl.ds(r0, nr)]))
    _rowcopy(sub, lambda r0, nr: pltpu.sync_copy(
        z_hbm.at[pl.ds(core * N + r0, nr)], z_s.at[pl.ds(r0, nr)]))
    plsc.subcore_barrier()

    base = sub * EPS
    rows = (rows0, rows1)
    semi = (semi0, semi1)
    semg = (semg0, semg1)

    def idx_start(c, p):
        off = base + c * CHUNK
        pltpu.async_copy(src_hbm.at[pl.ds(off, CHUNK)], sb.at[p], semi[p])
        pltpu.async_copy(dst_hbm.at[pl.ds(off, CHUNK)], db.at[p], semi[p])

    def idx_wait(p):
        pltpu.make_async_copy(src_hbm.at[pl.ds(0, CHUNK)], sb.at[p], semi[p]).wait()
        pltpu.make_async_copy(dst_hbm.at[pl.ds(0, CHUNK)], db.at[p], semi[p]).wait()

    def transform(p):
        for j in range(CHUNK // 16):
            sl = pl.ds(j * 16, 16)
            s16 = sb[p, sl]
            d16 = db[p, sl]
            gb[p, sl] = s16
            wb[p, sl] = jnp.where(s16 == d16, GARB, d16)

    def gather_start(p):
        pltpu.async_copy(z_s.at[gb.at[p]], rows[p], semg[p])

    def gather_wait(p):
        pltpu.make_async_copy(z_s.at[gb.at[p]], rows[p], semg[p]).wait()

    def scatter(p):
        pltpu.sync_copy(rows[p], acc.at[wb.at[p]], add=True)

    idx_start(0, 0)
    idx_start(1, 1)
    idx_wait(0)
    transform(0)
    gather_start(0)
    idx_start(2, 0)

    def body(k2, carry):
        idx_wait(1)
        transform(1)
        gather_start(1)

        @pl.when(k2 < NK2 - 1)
        def _():
            idx_start(2 * k2 + 3, 1)

        gather_wait(0)
        scatter(0)

        @pl.when(k2 < NK2 - 1)
        def _():
            idx_wait(0)
            transform(0)
            gather_start(0)

        @pl.when(k2 < NK2 - 2)
        def _():
            idx_start(2 * k2 + 4, 0)

        gather_wait(1)
        scatter(1)
        return carry

    lax.fori_loop(0, NK2, body, 0)
    plsc.subcore_barrier()
    _rowcopy(sub, lambda r0, nr: pltpu.sync_copy(
        acc.at[pl.ds(r0, nr)], out_hbm.at[pl.ds(core * N + r0, nr)]))


@functools.partial(
    pl.kernel,
    out_type=jax.ShapeDtypeStruct((NCORE * N, 16), jnp.float32),
    mesh=_MESH,
    compiler_params=pltpu.CompilerParams(use_tc_tiling_on_sc=False),
    scratch_types=[
        pltpu.VMEM((2, CHUNK), jnp.int32),
        pltpu.VMEM((2, CHUNK), jnp.int32),
        pltpu.VMEM((2, CHUNK), jnp.int32),
        pltpu.VMEM((CHUNK, 16), jnp.float32),
        pltpu.VMEM_SHARED((ACC_ROWS, 16), jnp.float32),
        pltpu.SemaphoreType.DMA,
        pltpu.SemaphoreType.DMA,
    ],
)
def _degrees(src_hbm, dst_hbm, init_hbm, out_hbm,
             sb, db, wb, ones, acc, semi0, semi1):
    core = lax.axis_index("c")
    sub = lax.axis_index("s")
    pltpu.sync_copy(init_hbm.at[pl.ds(0, CHUNK)], ones)
    _rowcopy(sub, lambda r0, nr: pltpu.sync_copy(
        init_hbm.at[pl.ds(r0, nr)], acc.at[pl.ds(r0, nr)]))
    plsc.subcore_barrier()

    base = sub * EPS
    semi = (semi0, semi1)

    def idx_start(c, p):
        off = base + c * CHUNK
        pltpu.async_copy(src_hbm.at[pl.ds(off, CHUNK)], sb.at[p], semi[p])
        pltpu.async_copy(dst_hbm.at[pl.ds(off, CHUNK)], db.at[p], semi[p])

    def idx_wait(p):
        pltpu.make_async_copy(src_hbm.at[pl.ds(0, CHUNK)], sb.at[p], semi[p]).wait()
        pltpu.make_async_copy(dst_hbm.at[pl.ds(0, CHUNK)], db.at[p], semi[p]).wait()

    def step(p):
        for j in range(CHUNK // 16):
            sl = pl.ds(j * 16, 16)
            s16 = sb[p, sl]
            d16 = db[p, sl]
            i16 = jnp.where(core == 0, s16, d16)
            wb[p, sl] = jnp.where(s16 == d16, GARB, i16)
        pltpu.sync_copy(ones, acc.at[wb.at[p]], add=True)

    idx_start(0, 0)
    idx_start(1, 1)

    def body(k2, carry):
        idx_wait(0)

        @pl.when(k2 < NK2 - 1)
        def _():
            idx_start(2 * k2 + 2, 0)

        step(0)
        idx_wait(1)

        @pl.when(k2 < NK2 - 1)
        def _():
            idx_start(2 * k2 + 3, 1)

        step(1)
        return carry

    lax.fori_loop(0, NK2, body, 0)
    plsc.subcore_barrier()
    _rowcopy(sub, lambda r0, nr: pltpu.sync_copy(
        acc.at[pl.ds(r0, nr)], out_hbm.at[pl.ds(core * N + r0, nr)]))


BN = 2000


def _pre_body(h_ref, ns_ref, w_ref, o_ref):
    o_ref[0] = jnp.dot(h_ref[0] * ns_ref[...], w_ref[...],
                       preferred_element_type=jnp.float32)


def _pre_call(h, ns, W):
    return pl.pallas_call(
        _pre_body,
        grid=(B, N // BN),
        in_specs=[
            pl.BlockSpec((1, BN, F), lambda c, i: (c, i, 0)),
            pl.BlockSpec((BN, 1), lambda c, i: (i, 0)),
            pl.BlockSpec((F, F), lambda c, i: (0, 0)),
        ],
        out_specs=pl.BlockSpec((1, BN, F), lambda c, i: (c, i, 0)),
        out_shape=jax.ShapeDtypeStruct((B, N, F), jnp.float32),
    )(h, ns, W)


def _mid_body(s_ref, zs_ref, nd_ref, b_ref, ns_ref, w_ref, o_ref):
    y = (s_ref[0] + zs_ref[0]) * nd_ref[...] + b_ref[...]
    y = jnp.where(y > 0, y, 0.01 * y)
    o_ref[0] = jnp.dot(y * ns_ref[...], w_ref[...],
                       preferred_element_type=jnp.float32)


def _mid_call(s, zs, nd, b, ns, W):
    Dout = W.shape[1]
    return pl.pallas_call(
        _mid_body,
        grid=(B, N // BN),
        in_specs=[
            pl.BlockSpec((1, BN, F), lambda c, i: (c, i, 0)),
            pl.BlockSpec((1, BN, F), lambda c, i: (c, i, 0)),
            pl.BlockSpec((BN, 1), lambda c, i: (i, 0)),
            pl.BlockSpec((1, F), lambda c, i: (0, 0)),
            pl.BlockSpec((BN, 1), lambda c, i: (i, 0)),
            pl.BlockSpec((F, Dout), lambda c, i: (0, 0)),
        ],
        out_specs=pl.BlockSpec((1, BN, Dout), lambda c, i: (c, i, 0)),
        out_shape=jax.ShapeDtypeStruct((B, N, Dout), jnp.float32),
    )(s, zs, nd, b, ns, W)


def _norm_body(deg_ref, ns_ref, nd_ref):
    ns_ref[...] = lax.rsqrt(deg_ref[0, :, 0:1])
    nd_ref[...] = lax.rsqrt(deg_ref[1, :, 0:1])


def _norm_call(deg):
    return pl.pallas_call(
        _norm_body,
        grid=(N // BN,),
        in_specs=[pl.BlockSpec((2, BN, 16), lambda i: (0, i, 0))],
        out_specs=[pl.BlockSpec((BN, 1), lambda i: (i, 0))] * 2,
        out_shape=[jax.ShapeDtypeStruct((N, 1), jnp.float32)] * 2,
    )(deg)


def _post3_body(s3_ref, h_ref, nd_ref, b3_ref, out_ref, hn_ref):
    o = s3_ref[0] * nd_ref[...] + b3_ref[...]
    out_ref[0] = o
    hn_ref[0] = jnp.concatenate([h_ref[0][:, 3:], o[:, :3]], axis=1)


def _post3_call(s3, h, nd, b3p):
    return pl.pallas_call(
        _post3_body,
        grid=(B, N // BN),
        in_specs=[
            pl.BlockSpec((1, BN, 16), lambda c, i: (c, i, 0)),
            pl.BlockSpec((1, BN, F), lambda c, i: (c, i, 0)),
            pl.BlockSpec((BN, 1), lambda c, i: (i, 0)),
            pl.BlockSpec((1, 16), lambda c, i: (0, 0)),
        ],
        out_specs=[
            pl.BlockSpec((1, BN, 16), lambda c, i: (c, i, 0)),
            pl.BlockSpec((1, BN, F), lambda c, i: (c, i, 0)),
        ],
        out_shape=[
            jax.ShapeDtypeStruct((B, N, 16), jnp.float32),
            jax.ShapeDtypeStruct((B, N, F), jnp.float32),
        ],
    )(s3, h, nd, b3p)


def kernel(edge_index, xx, output_length, W1, b1, W2, b2, W3, b3):
    src = edge_index[0].astype(jnp.int32)
    dst = edge_index[1].astype(jnp.int32)
    padn = E_PAD - E
    srcp = jnp.concatenate([src, jnp.zeros((padn,), jnp.int32)])
    dstp = jnp.concatenate([dst, jnp.zeros((padn,), jnp.int32)])

    init16 = jnp.tile(
        (lax.iota(jnp.int32, 16) == 0).astype(jnp.float32)[None, :], (N, 1))
    zeros64 = jnp.zeros((RPS_A, 64), jnp.float32)

    deg = _degrees(srcp, dstp, init16)
    ns, nd = _norm_call(deg.reshape(NCORE, N, 16))

    b1r = b1.reshape(1, F)
    b2r = b2.reshape(1, F)
    W3p = jnp.pad(W3, ((0, 0), (0, 13)))
    b3p = jnp.pad(b3, (0, 13)).reshape(1, 16)

    def agg_full(z):
        z64 = z.reshape(2 * NCORE * N, 64)
        s64 = _agg128(z64, srcp, dstp, zeros64)
        return s64.reshape(B, N, F)

    h = xx
    outs = []
    for _ in range(2):
        z1 = _pre_call(h, ns, W1)
        s1 = agg_full(z1)
        z2 = _mid_call(s1, z1, nd, b1r, ns, W2)
        s2 = agg_full(z2)
        p = _mid_call(s2, z2, nd, b2r, ns, W3p)
        s3 = _agg16(p.reshape(NCORE * N, 16), srcp, dstp)
        out_t, h = _post3_call(s3.reshape(B, N, 16), h, nd, b3p)
        outs.append(out_t[:, :, :3])
    res = jnp.stack(outs, axis=2)
    res = res * (jnp.asarray(output_length) // 2).astype(res.dtype)
    return res


# speedup vs baseline: 1.1807x; 1.1807x over previous
import functools

import jax
import jax.numpy as jnp
from jax import lax
from jax.experimental import pallas as pl
from jax.experimental.pallas import tpu as pltpu
from jax.experimental.pallas import tpu_sc as plsc

N = 10000
F = 128
E = 320000
B = 2
NSUB = 16
NCORE = 2
CHUNK = 128
NCHUNK = 160
NK2 = NCHUNK // 2
NK4 = NCHUNK // 4
EPS = NCHUNK * CHUNK
E_PAD = EPS * NSUB
GARB = N
ACC_ROWS = N + 8
RPN = N // NSUB
NSTG = 5
ZS_ROWS = 15 * RPN + NSTG * CHUNK
RPS_A = 632
RPS_LAST = N - 15 * RPS_A

_MESH = plsc.VectorSubcoreMesh(core_axis_name="c", subcore_axis_name="s")


def _rowcopy(sub, copy_fn):
    @pl.when(sub < NSUB - 1)
    def _():
        copy_fn(sub * RPS_A, RPS_A)

    @pl.when(sub == NSUB - 1)
    def _():
        copy_fn((NSUB - 1) * RPS_A, RPS_LAST)


def _agg128_kernel():
    @functools.partial(
        pl.kernel,
        out_type=jax.ShapeDtypeStruct((2 * NCORE * N, 64), jnp.float32),
        mesh=_MESH,
        compiler_params=pltpu.CompilerParams(use_tc_tiling_on_sc=False),
        scratch_types=[
            pltpu.VMEM((4, CHUNK), jnp.int32),
            pltpu.VMEM((4, CHUNK), jnp.int32),
            pltpu.VMEM((4, CHUNK), jnp.int32),
            pltpu.VMEM((4, CHUNK), jnp.int32),
            pltpu.VMEM((CHUNK,), jnp.int32),
            pltpu.VMEM((CHUNK,), jnp.int32),
            pltpu.VMEM((CHUNK, 64), jnp.float32),
            pltpu.VMEM((CHUNK, 64), jnp.float32),
            pltpu.VMEM((CHUNK, 64), jnp.float32),
            pltpu.VMEM((CHUNK, 64), jnp.float32),
            pltpu.VMEM_SHARED((ACC_ROWS, 64), jnp.float32),
            pltpu.VMEM_SHARED((ZS_ROWS, 64), jnp.float32),
        ] + [pltpu.SemaphoreType.DMA] * 12,
    )
    def agg(z64, src_hbm, dst_hbm, zero_hbm, out64,
            sb, db, gb, wb, stg, og, rows0, rows1, rows2, rows3, acc, z_s,
            *sems):
        core = lax.axis_index("c")
        sub = lax.axis_index("s")
        base = sub * EPS
        rows = (rows0, rows1, rows2, rows3)
        semi = sems[0:4]
        semg = sems[4:8]
        sema = sems[8:12]
        iota = lax.iota(jnp.int32, 16)

        def idx_start(c, p):
            off = base + c * CHUNK
            pltpu.async_copy(src_hbm.at[pl.ds(off, CHUNK)], sb.at[p], semi[p])
            pltpu.async_copy(dst_hbm.at[pl.ds(off, CHUNK)], db.at[p], semi[p])

        def idx_wait(p):
            pltpu.make_async_copy(src_hbm.at[pl.ds(0, CHUNK)], sb.at[p], semi[p]).wait()
            pltpu.make_async_copy(dst_hbm.at[pl.ds(0, CHUNK)], db.at[p], semi[p]).wait()

        def transform(p):
            for j in range(CHUNK // 16):
                sl = pl.ds(j * 16, 16)
                s16 = sb[p, sl]
                d16 = db[p, sl]
                gb[p, sl] = s16
                wb[p, sl] = jnp.where(s16 == d16, GARB, d16)

        def gather_start(p):
            pltpu.async_copy(z_s.at[gb.at[p]], rows[p], semg[p])

        def gather_wait(p):
            pltpu.make_async_copy(z_s.at[gb.at[p]], rows[p], semg[p]).wait()

        def scatter_start(p):
            pltpu.async_copy(rows[p], acc.at[wb.at[p]], sema[p], add=True)

        def scatter_wait(p):
            pltpu.make_async_copy(rows[p], acc.at[wb.at[p]], sema[p]).wait()

        for h in range(2):
            hoff = 2 * core * N + h
            _rowcopy(sub, lambda r0, nr: pltpu.sync_copy(
                zero_hbm.at[pl.ds(0, nr)], acc.at[pl.ds(r0, nr)]))
            for k in range(NSTG):
                r0 = sub * RPN + k * CHUNK
                for j in range(CHUNK // 16):
                    m = jnp.minimum(r0 + j * 16 + iota, N - 1)
                    stg[pl.ds(j * 16, 16)] = 2 * m + hoff
                pltpu.sync_copy(z64.at[stg], rows0)
                pltpu.sync_copy(rows0, z_s.at[pl.ds(r0, CHUNK)])
            plsc.subcore_barrier()

            for s in range(4):
                idx_start(s, s)

            def body(k, carry):
                for s in range(4):
                    @pl.when(k > 0)
                    def _(s=s):
                        scatter_wait(s)
                    idx_wait(s)
                    transform(s)
                    gather_start(s)

                    @pl.when(k < NK4 - 1)
                    def _(s=s):
                        idx_start(4 * k + 4 + s, s)

                for s in range(4):
                    gather_wait(s)
                    scatter_start(s)
                return carry

            lax.fori_loop(0, NK4, body, 0)
            for s in range(4):
                scatter_wait(s)
            plsc.subcore_barrier()

            for k in range(NSTG):
                r0 = sub * RPN + k * CHUNK
                for j in range(CHUNK // 16):
                    m = jnp.minimum(r0 + j * 16 + iota, N - 1)
                    stg[pl.ds(j * 16, 16)] = m
                    og[pl.ds(j * 16, 16)] = 2 * m + hoff
                pltpu.sync_copy(acc.at[stg], rows0)
                pltpu.sync_copy(rows0, out64.at[og])
            plsc.subcore_barrier()

    return agg


_agg128 = _agg128_kernel()


@functools.partial(
    pl.kernel,
    out_type=jax.ShapeDtypeStruct((NCORE * N, 16), jnp.float32),
    mesh=_MESH,
    compiler_params=pltpu.CompilerParams(use_tc_tiling_on_sc=False),
    scratch_types=[
        pltpu.VMEM((2, CHUNK), jnp.int32),
        pltpu.VMEM((2, CHUNK), jnp.int32),
        pltpu.VMEM((2, CHUNK), jnp.int32),
        pltpu.VMEM((2, CHUNK), jnp.int32),
        pltpu.VMEM((CHUNK, 16), jnp.float32),
        pltpu.VMEM((CHUNK, 16), jnp.float32),
        pltpu.VMEM_SHARED((ACC_ROWS, 16), jnp.float32),
        pltpu.VMEM_SHARED((N, 16), jnp.float32),
        pltpu.SemaphoreType.DMA,
        pltpu.SemaphoreType.DMA,
        pltpu.SemaphoreType.DMA,
        pltpu.SemaphoreType.DMA,
    ],
)
def _agg16(z_hbm, src_hbm, dst_hbm, out_hbm,
           sb, db, gb, wb, rows0, rows1, acc, z_s,
           semi0, semi1, semg0, semg1):
    core = lax.axis_index("c")
    sub = lax.axis_index("s")
    _rowcopy(sub, lambda r0, nr: pltpu.sync_copy(
        z_hbm.at[pl.ds(core * N + r0, nr)], acc.at[pl.ds(r0, nr)]))
    _rowcopy(sub, lambda r0, nr: pltpu.sync_copy(
        z_hbm.at[pl.ds(core * N + r0, nr)], z_s.at[pl.ds(r0, nr)]))
    plsc.subcore_barrier()

    base = sub * EPS
    rows = (rows0, rows1)
    semi = (semi0, semi1)
    semg = (semg0, semg1)

    def idx_start(c, p):
        off = base + c * CHUNK
        pltpu.async_copy(src_hbm.at[pl.ds(off, CHUNK)], sb.at[p], semi[p])
        pltpu.async_copy(dst_hbm.at[pl.ds(off, CHUNK)], db.at[p], semi[p])

    def idx_wait(p):
        pltpu.make_async_copy(src_hbm.at[pl.ds(0, CHUNK)], sb.at[p], semi[p]).wait()
        pltpu.make_async_copy(dst_hbm.at[pl.ds(0, CHUNK)], db.at[p], semi[p]).wait()

    def transform(p):
        for j in range(CHUNK // 16):
            sl = pl.ds(j * 16, 16)
            s16 = sb[p, sl]
            d16 = db[p, sl]
            gb[p, sl] = s16
            wb[p, sl] = jnp.where(s16 == d16, GARB, d16)

    def gather_start(p):
        pltpu.async_copy(z_s.at[gb.at[p]], rows[p], semg[p])

    def gather_wait(p):
        pltpu.make_async_copy(z_s.at[gb.at[p]], rows[p], semg[p]).wait()

    def scatter(p):
        pltpu.sync_copy(rows[p], acc.at[wb.at[p]], add=True)

    idx_start(0, 0)
    idx_start(1, 1)
    idx_wait(0)
    transform(0)
    gather_start(0)
    idx_start(2, 0)

    def body(k2, carry):
        idx_wait(1)
        transform(1)
        gather_start(1)

        @pl.when(k2 < NK2 - 1)
        def _():
            idx_start(2 * k2 + 3, 1)

        gather_wait(0)
        scatter(0)

        @pl.when(k2 < NK2 - 1)
        def _():
            idx_wait(0)
            transform(0)
            gather_start(0)

        @pl.when(k2 < NK2 - 2)
        def _():
            idx_start(2 * k2 + 4, 0)

        gather_wait(1)
        scatter(1)
        return carry

    lax.fori_loop(0, NK2, body, 0)
    plsc.subcore_barrier()
    _rowcopy(sub, lambda r0, nr: pltpu.sync_copy(
        acc.at[pl.ds(r0, nr)], out_hbm.at[pl.ds(core * N + r0, nr)]))


@functools.partial(
    pl.kernel,
    out_type=jax.ShapeDtypeStruct((NCORE * N, 16), jnp.float32),
    mesh=_MESH,
    compiler_params=pltpu.CompilerParams(use_tc_tiling_on_sc=False),
    scratch_types=[
        pltpu.VMEM((2, CHUNK), jnp.int32),
        pltpu.VMEM((2, CHUNK), jnp.int32),
        pltpu.VMEM((2, CHUNK), jnp.int32),
        pltpu.VMEM((CHUNK, 16), jnp.float32),
        pltpu.VMEM_SHARED((ACC_ROWS, 16), jnp.float32),
        pltpu.SemaphoreType.DMA,
        pltpu.SemaphoreType.DMA,
    ],
)
def _degrees(src_hbm, dst_hbm, init_hbm, out_hbm,
             sb, db, wb, ones, acc, semi0, semi1):
    core = lax.axis_index("c")
    sub = lax.axis_index("s")
    pltpu.sync_copy(init_hbm.at[pl.ds(0, CHUNK)], ones)
    _rowcopy(sub, lambda r0, nr: pltpu.sync_copy(
        init_hbm.at[pl.ds(r0, nr)], acc.at[pl.ds(r0, nr)]))
    plsc.subcore_barrier()

    base = sub * EPS
    semi = (semi0, semi1)

    def idx_start(c, p):
        off = base + c * CHUNK
        pltpu.async_copy(src_hbm.at[pl.ds(off, CHUNK)], sb.at[p], semi[p])
        pltpu.async_copy(dst_hbm.at[pl.ds(off, CHUNK)], db.at[p], semi[p])

    def idx_wait(p):
        pltpu.make_async_copy(src_hbm.at[pl.ds(0, CHUNK)], sb.at[p], semi[p]).wait()
        pltpu.make_async_copy(dst_hbm.at[pl.ds(0, CHUNK)], db.at[p], semi[p]).wait()

    def step(p):
        for j in range(CHUNK // 16):
            sl = pl.ds(j * 16, 16)
            s16 = sb[p, sl]
            d16 = db[p, sl]
            i16 = jnp.where(core == 0, s16, d16)
            wb[p, sl] = jnp.where(s16 == d16, GARB, i16)
        pltpu.sync_copy(ones, acc.at[wb.at[p]], add=True)

    idx_start(0, 0)
    idx_start(1, 1)

    def body(k2, carry):
        idx_wait(0)

        @pl.when(k2 < NK2 - 1)
        def _():
            idx_start(2 * k2 + 2, 0)

        step(0)
        idx_wait(1)

        @pl.when(k2 < NK2 - 1)
        def _():
            idx_start(2 * k2 + 3, 1)

        step(1)
        return carry

    lax.fori_loop(0, NK2, body, 0)
    plsc.subcore_barrier()
    _rowcopy(sub, lambda r0, nr: pltpu.sync_copy(
        acc.at[pl.ds(r0, nr)], out_hbm.at[pl.ds(core * N + r0, nr)]))


BN = 2000


def _pre_body(h_ref, ns_ref, w_ref, o_ref):
    o_ref[0] = jnp.dot(h_ref[0] * ns_ref[...], w_ref[...],
                       preferred_element_type=jnp.float32)


def _pre_call(h, ns, W):
    return pl.pallas_call(
        _pre_body,
        grid=(B, N // BN),
        in_specs=[
            pl.BlockSpec((1, BN, F), lambda c, i: (c, i, 0)),
            pl.BlockSpec((BN, 1), lambda c, i: (i, 0)),
            pl.BlockSpec((F, F), lambda c, i: (0, 0)),
        ],
        out_specs=pl.BlockSpec((1, BN, F), lambda c, i: (c, i, 0)),
        out_shape=jax.ShapeDtypeStruct((B, N, F), jnp.float32),
    )(h, ns, W)


def _mid_body(s_ref, zs_ref, nd_ref, b_ref, ns_ref, w_ref, o_ref):
    y = (s_ref[0] + zs_ref[0]) * nd_ref[...] + b_ref[...]
    y = jnp.where(y > 0, y, 0.01 * y)
    o_ref[0] = jnp.dot(y * ns_ref[...], w_ref[...],
                       preferred_element_type=jnp.float32)


def _mid_call(s, zs, nd, b, ns, W):
    Dout = W.shape[1]
    return pl.pallas_call(
        _mid_body,
        grid=(B, N // BN),
        in_specs=[
            pl.BlockSpec((1, BN, F), lambda c, i: (c, i, 0)),
            pl.BlockSpec((1, BN, F), lambda c, i: (c, i, 0)),
            pl.BlockSpec((BN, 1), lambda c, i: (i, 0)),
            pl.BlockSpec((1, F), lambda c, i: (0, 0)),
            pl.BlockSpec((BN, 1), lambda c, i: (i, 0)),
            pl.BlockSpec((F, Dout), lambda c, i: (0, 0)),
        ],
        out_specs=pl.BlockSpec((1, BN, Dout), lambda c, i: (c, i, 0)),
        out_shape=jax.ShapeDtypeStruct((B, N, Dout), jnp.float32),
    )(s, zs, nd, b, ns, W)


def _norm_body(deg_ref, ns_ref, nd_ref):
    ns_ref[...] = lax.rsqrt(deg_ref[0, :, 0:1])
    nd_ref[...] = lax.rsqrt(deg_ref[1, :, 0:1])


def _norm_call(deg):
    return pl.pallas_call(
        _norm_body,
        grid=(N // BN,),
        in_specs=[pl.BlockSpec((2, BN, 16), lambda i: (0, i, 0))],
        out_specs=[pl.BlockSpec((BN, 1), lambda i: (i, 0))] * 2,
        out_shape=[jax.ShapeDtypeStruct((N, 1), jnp.float32)] * 2,
    )(deg)


def _post3_body(s3_ref, h_ref, nd_ref, b3_ref, out_ref, hn_ref):
    o = s3_ref[0] * nd_ref[...] + b3_ref[...]
    out_ref[0] = o
    hn_ref[0] = jnp.concatenate([h_ref[0][:, 3:], o[:, :3]], axis=1)


def _post3_call(s3, h, nd, b3p):
    return pl.pallas_call(
        _post3_body,
        grid=(B, N // BN),
        in_specs=[
            pl.BlockSpec((1, BN, 16), lambda c, i: (c, i, 0)),
            pl.BlockSpec((1, BN, F), lambda c, i: (c, i, 0)),
            pl.BlockSpec((BN, 1), lambda c, i: (i, 0)),
            pl.BlockSpec((1, 16), lambda c, i: (0, 0)),
        ],
        out_specs=[
            pl.BlockSpec((1, BN, 16), lambda c, i: (c, i, 0)),
            pl.BlockSpec((1, BN, F), lambda c, i: (c, i, 0)),
        ],
        out_shape=[
            jax.ShapeDtypeStruct((B, N, 16), jnp.float32),
            jax.ShapeDtypeStruct((B, N, F), jnp.float32),
        ],
    )(s3, h, nd, b3p)


def kernel(edge_index, xx, output_length, W1, b1, W2, b2, W3, b3):
    src = edge_index[0].astype(jnp.int32)
    dst = edge_index[1].astype(jnp.int32)
    padn = E_PAD - E
    srcp = jnp.concatenate([src, jnp.zeros((padn,), jnp.int32)])
    dstp = jnp.concatenate([dst, jnp.zeros((padn,), jnp.int32)])

    init16 = jnp.tile(
        (lax.iota(jnp.int32, 16) == 0).astype(jnp.float32)[None, :], (N, 1))
    zeros64 = jnp.zeros((RPS_A, 64), jnp.float32)

    deg = _degrees(srcp, dstp, init16)
    ns, nd = _norm_call(deg.reshape(NCORE, N, 16))

    b1r = b1.reshape(1, F)
    b2r = b2.reshape(1, F)
    W3p = jnp.pad(W3, ((0, 0), (0, 13)))
    b3p = jnp.pad(b3, (0, 13)).reshape(1, 16)

    def agg_full(z):
        z64 = z.reshape(2 * NCORE * N, 64)
        s64 = _agg128(z64, srcp, dstp, zeros64)
        return s64.reshape(B, N, F)

    h = xx
    outs = []
    for _ in range(2):
        z1 = _pre_call(h, ns, W1)
        s1 = agg_full(z1)
        z2 = _mid_call(s1, z1, nd, b1r, ns, W2)
        s2 = agg_full(z2)
        p = _mid_call(s2, z2, nd, b2r, ns, W3p)
        s3 = _agg16(p.reshape(NCORE * N, 16), srcp, dstp)
        out_t, h = _post3_call(s3.reshape(B, N, 16), h, nd, b3p)
        outs.append(out_t[:, :, :3])
    res = jnp.stack(outs, axis=2)
    res = res * (jnp.asarray(output_length) // 2).astype(res.dtype)
    return res


# speedup vs baseline: 1.3856x; 1.1736x over previous
import functools

import jax
import jax.numpy as jnp
from jax import lax
from jax.experimental import pallas as pl
from jax.experimental.pallas import tpu as pltpu
from jax.experimental.pallas import tpu_sc as plsc

N = 10000
F = 128
E = 320000
B = 2
NSUB = 16
NCORE = 2
CHUNK = 128
NCHUNK = 160
NK2 = NCHUNK // 2
EPS = NCHUNK * CHUNK
E_PAD = EPS * NSUB
GARB = N
ACC_ROWS = N + 8
RPN = N // NSUB
NSTG = 5
ZS_ROWS = 15 * RPN + NSTG * CHUNK
RPS_A = 632
RPS_LAST = N - 15 * RPS_A

_MESH = plsc.VectorSubcoreMesh(core_axis_name="c", subcore_axis_name="s")


def _rowcopy(sub, copy_fn):
    @pl.when(sub < NSUB - 1)
    def _():
        copy_fn(sub * RPS_A, RPS_A)

    @pl.when(sub == NSUB - 1)
    def _():
        copy_fn((NSUB - 1) * RPS_A, RPS_LAST)


def _agg128_kernel():
    @functools.partial(
        pl.kernel,
        out_type=jax.ShapeDtypeStruct((2 * NCORE * N, 64), jnp.float32),
        mesh=_MESH,
        compiler_params=pltpu.CompilerParams(use_tc_tiling_on_sc=False),
        scratch_types=[
            pltpu.VMEM((2, CHUNK), jnp.int32),
            pltpu.VMEM((2, CHUNK), jnp.int32),
            pltpu.VMEM((2, CHUNK), jnp.int32),
            pltpu.VMEM((2, CHUNK), jnp.int32),
            pltpu.VMEM((CHUNK,), jnp.int32),
            pltpu.VMEM((CHUNK,), jnp.int32),
            pltpu.VMEM((CHUNK, 64), jnp.float32),
            pltpu.VMEM((CHUNK, 64), jnp.float32),
            pltpu.VMEM_SHARED((ACC_ROWS, 64), jnp.float32),
            pltpu.VMEM_SHARED((ZS_ROWS, 64), jnp.float32),
            pltpu.SemaphoreType.DMA,
            pltpu.SemaphoreType.DMA,
            pltpu.SemaphoreType.DMA,
            pltpu.SemaphoreType.DMA,
        ],
    )
    def agg(z64, src_hbm, dst_hbm, zero_hbm, out64,
            sb, db, gb, wb, stg, og, rows0, rows1, acc, z_s,
            semi0, semi1, semg0, semg1):
        core = lax.axis_index("c")
        sub = lax.axis_index("s")
        base = sub * EPS
        rows = (rows0, rows1)
        semi = (semi0, semi1)
        semg = (semg0, semg1)
        iota = lax.iota(jnp.int32, 16)

        def idx_start(c, p):
            off = base + c * CHUNK
            pltpu.async_copy(src_hbm.at[pl.ds(off, CHUNK)], sb.at[p], semi[p])
            pltpu.async_copy(dst_hbm.at[pl.ds(off, CHUNK)], db.at[p], semi[p])

        def idx_wait(p):
            pltpu.make_async_copy(src_hbm.at[pl.ds(0, CHUNK)], sb.at[p], semi[p]).wait()
            pltpu.make_async_copy(dst_hbm.at[pl.ds(0, CHUNK)], db.at[p], semi[p]).wait()

        def transform(p):
            for j in range(CHUNK // 16):
                sl = pl.ds(j * 16, 16)
                s16 = sb[p, sl]
                d16 = db[p, sl]
                gb[p, sl] = s16
                wb[p, sl] = jnp.where(s16 == d16, GARB, d16)

        def gather_start(p):
            pltpu.async_copy(z_s.at[gb.at[p]], rows[p], semg[p])

        def gather_wait(p):
            pltpu.make_async_copy(z_s.at[gb.at[p]], rows[p], semg[p]).wait()

        def scatter(p):
            pltpu.sync_copy(rows[p], acc.at[wb.at[p]], add=True)

        for h in range(2):
            hoff = 2 * core * N + h
            _rowcopy(sub, lambda r0, nr: pltpu.sync_copy(
                zero_hbm.at[pl.ds(0, nr)], acc.at[pl.ds(r0, nr)]))
            def stage_copy(k, p):
                pltpu.make_async_copy(z64.at[gb.at[p]], rows[p], semg[p]).wait()
                pltpu.sync_copy(rows[p], z_s.at[pl.ds(sub * RPN + k * CHUNK, CHUNK)])

            for k in range(NSTG):
                p = k % 2
                if k >= 2:
                    stage_copy(k - 2, p)
                r0 = sub * RPN + k * CHUNK
                for j in range(CHUNK // 16):
                    m = jnp.minimum(r0 + j * 16 + iota, N - 1)
                    gb[p, pl.ds(j * 16, 16)] = 2 * m + hoff
                pltpu.async_copy(z64.at[gb.at[p]], rows[p], semg[p])
            for k in range(NSTG - 2, NSTG):
                stage_copy(k, k % 2)
            plsc.subcore_barrier()

            idx_start(0, 0)
            idx_start(1, 1)
            idx_wait(0)
            transform(0)
            gather_start(0)
            idx_start(2, 0)

            def body(k2, carry):
                idx_wait(1)
                transform(1)
                gather_start(1)

                @pl.when(k2 < NK2 - 1)
                def _():
                    idx_start(2 * k2 + 3, 1)

                gather_wait(0)
                scatter(0)

                @pl.when(k2 < NK2 - 1)
                def _():
                    idx_wait(0)
                    transform(0)
                    gather_start(0)

                @pl.when(k2 < NK2 - 2)
                def _():
                    idx_start(2 * k2 + 4, 0)

                gather_wait(1)
                scatter(1)
                return carry

            lax.fori_loop(0, NK2, body, 0)
            plsc.subcore_barrier()

            def wout_copy(p):
                pltpu.make_async_copy(acc.at[gb.at[p]], rows[p], semg[p]).wait()
                pltpu.sync_copy(rows[p], out64.at[wb.at[p]])

            for k in range(NSTG):
                p = k % 2
                if k >= 2:
                    wout_copy(p)
                r0 = sub * RPN + k * CHUNK
                for j in range(CHUNK // 16):
                    m = jnp.minimum(r0 + j * 16 + iota, N - 1)
                    gb[p, pl.ds(j * 16, 16)] = m
                    wb[p, pl.ds(j * 16, 16)] = 2 * m + hoff
                pltpu.async_copy(acc.at[gb.at[p]], rows[p], semg[p])
            for k in range(NSTG - 2, NSTG):
                wout_copy(k % 2)
            plsc.subcore_barrier()

    return agg


_agg128 = _agg128_kernel()


@functools.partial(
    pl.kernel,
    out_type=jax.ShapeDtypeStruct((NCORE * N, 16), jnp.float32),
    mesh=_MESH,
    compiler_params=pltpu.CompilerParams(use_tc_tiling_on_sc=False),
    scratch_types=[
        pltpu.VMEM((2, CHUNK), jnp.int32),
        pltpu.VMEM((2, CHUNK), jnp.int32),
        pltpu.VMEM((2, CHUNK), jnp.int32),
        pltpu.VMEM((2, CHUNK), jnp.int32),
        pltpu.VMEM((CHUNK, 16), jnp.float32),
        pltpu.VMEM((CHUNK, 16), jnp.float32),
        pltpu.VMEM_SHARED((ACC_ROWS, 16), jnp.float32),
        pltpu.VMEM_SHARED((N, 16), jnp.float32),
        pltpu.SemaphoreType.DMA,
        pltpu.SemaphoreType.DMA,
        pltpu.SemaphoreType.DMA,
        pltpu.SemaphoreType.DMA,
    ],
)
def _agg16(z_hbm, src_hbm, dst_hbm, out_hbm,
           sb, db, gb, wb, rows0, rows1, acc, z_s,
           semi0, semi1, semg0, semg1):
    core = lax.axis_index("c")
    sub = lax.axis_index("s")
    _rowcopy(sub, lambda r0, nr: pltpu.sync_copy(
        z_hbm.at[pl.ds(core * N + r0, nr)], acc.at[pl.ds(r0, nr)]))
    _rowcopy(sub, lambda r0, nr: pltpu.sync_copy(
        z_hbm.at[pl.ds(core * N + r0, nr)], z_s.at[pl.ds(r0, nr)]))
    plsc.subcore_barrier()

    base = sub * EPS
    rows = (rows0, rows1)
    semi = (semi0, semi1)
    semg = (semg0, semg1)

    def idx_start(c, p):
        off = base + c * CHUNK
        pltpu.async_copy(src_hbm.at[pl.ds(off, CHUNK)], sb.at[p], semi[p])
        pltpu.async_copy(dst_hbm.at[pl.ds(off, CHUNK)], db.at[p], semi[p])

    def idx_wait(p):
        pltpu.make_async_copy(src_hbm.at[pl.ds(0, CHUNK)], sb.at[p], semi[p]).wait()
        pltpu.make_async_copy(dst_hbm.at[pl.ds(0, CHUNK)], db.at[p], semi[p]).wait()

    def transform(p):
        for j in range(CHUNK // 16):
            sl = pl.ds(j * 16, 16)
            s16 = sb[p, sl]
            d16 = db[p, sl]
            gb[p, sl] = s16
            wb[p, sl] = jnp.where(s16 == d16, GARB, d16)

    def gather_start(p):
        pltpu.async_copy(z_s.at[gb.at[p]], rows[p], semg[p])

    def gather_wait(p):
        pltpu.make_async_copy(z_s.at[gb.at[p]], rows[p], semg[p]).wait()

    def scatter(p):
        pltpu.sync_copy(rows[p], acc.at[wb.at[p]], add=True)

    idx_start(0, 0)
    idx_start(1, 1)
    idx_wait(0)
    transform(0)
    gather_start(0)
    idx_start(2, 0)

    def body(k2, carry):
        idx_wait(1)
        transform(1)
        gather_start(1)

        @pl.when(k2 < NK2 - 1)
        def _():
            idx_start(2 * k2 + 3, 1)

        gather_wait(0)
        scatter(0)

        @pl.when(k2 < NK2 - 1)
        def _():
            idx_wait(0)
            transform(0)
            gather_start(0)

        @pl.when(k2 < NK2 - 2)
        def _():
            idx_start(2 * k2 + 4, 0)

        gather_wait(1)
        scatter(1)
        return carry

    lax.fori_loop(0, NK2, body, 0)
    plsc.subcore_barrier()
    _rowcopy(sub, lambda r0, nr: pltpu.sync_copy(
        acc.at[pl.ds(r0, nr)], out_hbm.at[pl.ds(core * N + r0, nr)]))


@functools.partial(
    pl.kernel,
    out_type=jax.ShapeDtypeStruct((NCORE * N, 16), jnp.float32),
    mesh=_MESH,
    compiler_params=pltpu.CompilerParams(use_tc_tiling_on_sc=False),
    scratch_types=[
        pltpu.VMEM((2, CHUNK), jnp.int32),
        pltpu.VMEM((2, CHUNK), jnp.int32),
        pltpu.VMEM((2, CHUNK), jnp.int32),
        pltpu.VMEM((CHUNK, 16), jnp.float32),
        pltpu.VMEM_SHARED((ACC_ROWS, 16), jnp.float32),
        pltpu.SemaphoreType.DMA,
        pltpu.SemaphoreType.DMA,
    ],
)
def _degrees(src_hbm, dst_hbm, init_hbm, out_hbm,
             sb, db, wb, ones, acc, semi0, semi1):
    core = lax.axis_index("c")
    sub = lax.axis_index("s")
    pltpu.sync_copy(init_hbm.at[pl.ds(0, CHUNK)], ones)
    _rowcopy(sub, lambda r0, nr: pltpu.sync_copy(
        init_hbm.at[pl.ds(r0, nr)], acc.at[pl.ds(r0, nr)]))
    plsc.subcore_barrier()

    base = sub * EPS
    semi = (semi0, semi1)

    def idx_start(c, p):
        off = base + c * CHUNK
        pltpu.async_copy(src_hbm.at[pl.ds(off, CHUNK)], sb.at[p], semi[p])
        pltpu.async_copy(dst_hbm.at[pl.ds(off, CHUNK)], db.at[p], semi[p])

    def idx_wait(p):
        pltpu.make_async_copy(src_hbm.at[pl.ds(0, CHUNK)], sb.at[p], semi[p]).wait()
        pltpu.make_async_copy(dst_hbm.at[pl.ds(0, CHUNK)], db.at[p], semi[p]).wait()

    def step(p):
        for j in range(CHUNK // 16):
            sl = pl.ds(j * 16, 16)
            s16 = sb[p, sl]
            d16 = db[p, sl]
            i16 = jnp.where(core == 0, s16, d16)
            wb[p, sl] = jnp.where(s16 == d16, GARB, i16)
        pltpu.sync_copy(ones, acc.at[wb.at[p]], add=True)

    idx_start(0, 0)
    idx_start(1, 1)

    def body(k2, carry):
        idx_wait(0)

        @pl.when(k2 < NK2 - 1)
        def _():
            idx_start(2 * k2 + 2, 0)

        step(0)
        idx_wait(1)

        @pl.when(k2 < NK2 - 1)
        def _():
            idx_start(2 * k2 + 3, 1)

        step(1)
        return carry

    lax.fori_loop(0, NK2, body, 0)
    plsc.subcore_barrier()
    _rowcopy(sub, lambda r0, nr: pltpu.sync_copy(
        acc.at[pl.ds(r0, nr)], out_hbm.at[pl.ds(core * N + r0, nr)]))


BN = 2000


def _pre_body(h_ref, ns_ref, w_ref, o_ref):
    o_ref[0] = jnp.dot(h_ref[0] * ns_ref[...], w_ref[...],
                       preferred_element_type=jnp.float32)


def _pre_call(h, ns, W):
    return pl.pallas_call(
        _pre_body,
        grid=(B, N // BN),
        in_specs=[
            pl.BlockSpec((1, BN, F), lambda c, i: (c, i, 0)),
            pl.BlockSpec((BN, 1), lambda c, i: (i, 0)),
            pl.BlockSpec((F, F), lambda c, i: (0, 0)),
        ],
        out_specs=pl.BlockSpec((1, BN, F), lambda c, i: (c, i, 0)),
        out_shape=jax.ShapeDtypeStruct((B, N, F), jnp.float32),
    )(h, ns, W)


def _mid_body(s_ref, zs_ref, nd_ref, b_ref, ns_ref, w_ref, o_ref):
    y = (s_ref[0] + zs_ref[0]) * nd_ref[...] + b_ref[...]
    y = jnp.where(y > 0, y, 0.01 * y)
    o_ref[0] = jnp.dot(y * ns_ref[...], w_ref[...],
                       preferred_element_type=jnp.float32)


def _mid_call(s, zs, nd, b, ns, W):
    Dout = W.shape[1]
    return pl.pallas_call(
        _mid_body,
        grid=(B, N // BN),
        in_specs=[
            pl.BlockSpec((1, BN, F), lambda c, i: (c, i, 0)),
            pl.BlockSpec((1, BN, F), lambda c, i: (c, i, 0)),
            pl.BlockSpec((BN, 1), lambda c, i: (i, 0)),
            pl.BlockSpec((1, F), lambda c, i: (0, 0)),
            pl.BlockSpec((BN, 1), lambda c, i: (i, 0)),
            pl.BlockSpec((F, Dout), lambda c, i: (0, 0)),
        ],
        out_specs=pl.BlockSpec((1, BN, Dout), lambda c, i: (c, i, 0)),
        out_shape=jax.ShapeDtypeStruct((B, N, Dout), jnp.float32),
    )(s, zs, nd, b, ns, W)


def _norm_body(deg_ref, ns_ref, nd_ref):
    ns_ref[...] = lax.rsqrt(deg_ref[0, :, 0:1])
    nd_ref[...] = lax.rsqrt(deg_ref[1, :, 0:1])


def _norm_call(deg):
    return pl.pallas_call(
        _norm_body,
        grid=(N // BN,),
        in_specs=[pl.BlockSpec((2, BN, 16), lambda i: (0, i, 0))],
        out_specs=[pl.BlockSpec((BN, 1), lambda i: (i, 0))] * 2,
        out_shape=[jax.ShapeDtypeStruct((N, 1), jnp.float32)] * 2,
    )(deg)


def _post3_body(s3_ref, h_ref, nd_ref, b3_ref, out_ref, hn_ref):
    o = s3_ref[0] * nd_ref[...] + b3_ref[...]
    out_ref[0] = o
    hn_ref[0] = jnp.concatenate([h_ref[0][:, 3:], o[:, :3]], axis=1)


def _post3_call(s3, h, nd, b3p):
    return pl.pallas_call(
        _post3_body,
        grid=(B, N // BN),
        in_specs=[
            pl.BlockSpec((1, BN, 16), lambda c, i: (c, i, 0)),
            pl.BlockSpec((1, BN, F), lambda c, i: (c, i, 0)),
            pl.BlockSpec((BN, 1), lambda c, i: (i, 0)),
            pl.BlockSpec((1, 16), lambda c, i: (0, 0)),
        ],
        out_specs=[
            pl.BlockSpec((1, BN, 16), lambda c, i: (c, i, 0)),
            pl.BlockSpec((1, BN, F), lambda c, i: (c, i, 0)),
        ],
        out_shape=[
            jax.ShapeDtypeStruct((B, N, 16), jnp.float32),
            jax.ShapeDtypeStruct((B, N, F), jnp.float32),
        ],
    )(s3, h, nd, b3p)


def kernel(edge_index, xx, output_length, W1, b1, W2, b2, W3, b3):
    src = edge_index[0].astype(jnp.int32)
    dst = edge_index[1].astype(jnp.int32)
    padn = E_PAD - E
    srcp = jnp.concatenate([src, jnp.zeros((padn,), jnp.int32)])
    dstp = jnp.concatenate([dst, jnp.zeros((padn,), jnp.int32)])

    init16 = jnp.tile(
        (lax.iota(jnp.int32, 16) == 0).astype(jnp.float32)[None, :], (N, 1))
    zeros64 = jnp.zeros((RPS_A, 64), jnp.float32)

    deg = _degrees(srcp, dstp, init16)
    ns, nd = _norm_call(deg.reshape(NCORE, N, 16))

    b1r = b1.reshape(1, F)
    b2r = b2.reshape(1, F)
    W3p = jnp.pad(W3, ((0, 0), (0, 13)))
    b3p = jnp.pad(b3, (0, 13)).reshape(1, 16)

    def agg_full(z):
        z64 = z.reshape(2 * NCORE * N, 64)
        s64 = _agg128(z64, srcp, dstp, zeros64)
        return s64.reshape(B, N, F)

    h = xx
    outs = []
    for _ in range(2):
        z1 = _pre_call(h, ns, W1)
        s1 = agg_full(z1)
        z2 = _mid_call(s1, z1, nd, b1r, ns, W2)
        s2 = agg_full(z2)
        p = _mid_call(s2, z2, nd, b2r, ns, W3p)
        s3 = _agg16(p.reshape(NCORE * N, 16), srcp, dstp)
        out_t, h = _post3_call(s3.reshape(B, N, 16), h, nd, b3p)
        outs.append(out_t[:, :, :3])
    res = jnp.stack(outs, axis=2)
    res = res * (jnp.asarray(output_length) // 2).astype(res.dtype)
    return res


# speedup vs baseline: 1.4061x; 1.0148x over previous
import functools

import jax
import jax.numpy as jnp
from jax import lax
from jax.experimental import pallas as pl
from jax.experimental.pallas import tpu as pltpu
from jax.experimental.pallas import tpu_sc as plsc

N = 10000
F = 128
E = 320000
B = 2
NSUB = 16
NCORE = 2
CHUNK = 128
NCHUNK = 160
NK2 = NCHUNK // 2
EPS = NCHUNK * CHUNK
E_PAD = EPS * NSUB
GARB = N
ACC_ROWS = N + 8
RPN = N // NSUB
NSTG = 5
ZS_ROWS = 15 * RPN + NSTG * CHUNK
RPS_A = 632
RPS_LAST = N - 15 * RPS_A

_MESH = plsc.VectorSubcoreMesh(core_axis_name="c", subcore_axis_name="s")


def _rowcopy(sub, copy_fn):
    @pl.when(sub < NSUB - 1)
    def _():
        copy_fn(sub * RPS_A, RPS_A)

    @pl.when(sub == NSUB - 1)
    def _():
        copy_fn((NSUB - 1) * RPS_A, RPS_LAST)


def _agg128_kernel():
    @functools.partial(
        pl.kernel,
        out_type=jax.ShapeDtypeStruct((2 * NCORE * N, 64), jnp.float32),
        mesh=_MESH,
        compiler_params=pltpu.CompilerParams(use_tc_tiling_on_sc=False),
        scratch_types=[
            pltpu.VMEM((2, CHUNK), jnp.int32),
            pltpu.VMEM((2, CHUNK), jnp.int32),
            pltpu.VMEM((2, CHUNK), jnp.int32),
            pltpu.VMEM((2, CHUNK), jnp.int32),
            pltpu.VMEM((CHUNK,), jnp.int32),
            pltpu.VMEM((CHUNK,), jnp.int32),
            pltpu.VMEM((CHUNK, 64), jnp.float32),
            pltpu.VMEM((CHUNK, 64), jnp.float32),
            pltpu.VMEM_SHARED((ACC_ROWS, 64), jnp.float32),
            pltpu.VMEM_SHARED((ZS_ROWS, 64), jnp.float32),
            pltpu.SemaphoreType.DMA,
            pltpu.SemaphoreType.DMA,
            pltpu.SemaphoreType.DMA,
            pltpu.SemaphoreType.DMA,
        ],
    )
    def agg(z64, src_hbm, dst_hbm, zero_hbm, out64,
            sb, db, gb, wb, stg, og, rows0, rows1, acc, z_s,
            semi0, semi1, semg0, semg1):
        core = lax.axis_index("c")
        sub = lax.axis_index("s")
        base = sub * EPS
        rows = (rows0, rows1)
        semi = (semi0, semi1)
        semg = (semg0, semg1)
        iota = lax.iota(jnp.int32, 16)

        def idx_start(c, p):
            off = base + c * CHUNK
            pltpu.async_copy(src_hbm.at[pl.ds(off, CHUNK)], sb.at[p], semi[p])
            pltpu.async_copy(dst_hbm.at[pl.ds(off, CHUNK)], db.at[p], semi[p])

        def idx_wait(p):
            pltpu.make_async_copy(src_hbm.at[pl.ds(0, CHUNK)], sb.at[p], semi[p]).wait()
            pltpu.make_async_copy(dst_hbm.at[pl.ds(0, CHUNK)], db.at[p], semi[p]).wait()

        def transform(p):
            for j in range(CHUNK // 16):
                sl = pl.ds(j * 16, 16)
                s16 = sb[p, sl]
                d16 = db[p, sl]
                gb[p, sl] = s16
                wb[p, sl] = jnp.where(s16 == d16, GARB, d16)

        def gather_start(p):
            pltpu.async_copy(z_s.at[gb.at[p]], rows[p], semg[p])

        def gather_wait(p):
            pltpu.make_async_copy(z_s.at[gb.at[p]], rows[p], semg[p]).wait()

        def scatter(p):
            pltpu.sync_copy(rows[p], acc.at[wb.at[p]], add=True)

        for h in range(2):
            hoff = 2 * core * N + h
            _rowcopy(sub, lambda r0, nr: pltpu.sync_copy(
                zero_hbm.at[pl.ds(0, nr)], acc.at[pl.ds(r0, nr)]))
            def stage_copy(k, p):
                pltpu.make_async_copy(z64.at[gb.at[p]], rows[p], semg[p]).wait()
                pltpu.sync_copy(rows[p], z_s.at[pl.ds(sub * RPN + k * CHUNK, CHUNK)])

            for k in range(NSTG):
                p = k % 2
                if k >= 2:
                    stage_copy(k - 2, p)
                r0 = sub * RPN + k * CHUNK
                for j in range(CHUNK // 16):
                    m = jnp.minimum(r0 + j * 16 + iota, N - 1)
                    gb[p, pl.ds(j * 16, 16)] = 2 * m + hoff
                pltpu.async_copy(z64.at[gb.at[p]], rows[p], semg[p])
            for k in range(NSTG - 2, NSTG):
                stage_copy(k, k % 2)
            plsc.subcore_barrier()

            idx_start(0, 0)
            idx_start(1, 1)
            idx_wait(0)
            transform(0)
            gather_start(0)
            idx_start(2, 0)

            def body(k2, carry):
                idx_wait(1)
                transform(1)
                gather_start(1)

                @pl.when(k2 < NK2 - 1)
                def _():
                    idx_start(2 * k2 + 3, 1)

                gather_wait(0)
                scatter(0)

                @pl.when(k2 < NK2 - 1)
                def _():
                    idx_wait(0)
                    transform(0)
                    gather_start(0)

                @pl.when(k2 < NK2 - 2)
                def _():
                    idx_start(2 * k2 + 4, 0)

                gather_wait(1)
                scatter(1)
                return carry

            lax.fori_loop(0, NK2, body, 0)
            plsc.subcore_barrier()

            def wout_copy(p):
                pltpu.make_async_copy(acc.at[gb.at[p]], rows[p], semg[p]).wait()
                pltpu.sync_copy(rows[p], out64.at[wb.at[p]])

            for k in range(NSTG):
                p = k % 2
                if k >= 2:
                    wout_copy(p)
                r0 = sub * RPN + k * CHUNK
                for j in range(CHUNK // 16):
                    m = jnp.minimum(r0 + j * 16 + iota, N - 1)
                    gb[p, pl.ds(j * 16, 16)] = m
                    wb[p, pl.ds(j * 16, 16)] = 2 * m + hoff
                pltpu.async_copy(acc.at[gb.at[p]], rows[p], semg[p])
            for k in range(NSTG - 2, NSTG):
                wout_copy(k % 2)
            plsc.subcore_barrier()

    return agg


_agg128 = _agg128_kernel()


@functools.partial(
    pl.kernel,
    out_type=jax.ShapeDtypeStruct((NCORE * N, 16), jnp.float32),
    mesh=_MESH,
    compiler_params=pltpu.CompilerParams(use_tc_tiling_on_sc=False),
    scratch_types=[
        pltpu.VMEM((2, CHUNK), jnp.int32),
        pltpu.VMEM((2, CHUNK), jnp.int32),
        pltpu.VMEM((2, CHUNK), jnp.int32),
        pltpu.VMEM((2, CHUNK), jnp.int32),
        pltpu.VMEM((CHUNK, 16), jnp.float32),
        pltpu.VMEM((CHUNK, 16), jnp.float32),
        pltpu.VMEM_SHARED((ACC_ROWS, 16), jnp.float32),
        pltpu.VMEM_SHARED((N, 16), jnp.float32),
        pltpu.SemaphoreType.DMA,
        pltpu.SemaphoreType.DMA,
        pltpu.SemaphoreType.DMA,
        pltpu.SemaphoreType.DMA,
    ],
)
def _agg16(z_hbm, src_hbm, dst_hbm, out_hbm,
           sb, db, gb, wb, rows0, rows1, acc, z_s,
           semi0, semi1, semg0, semg1):
    core = lax.axis_index("c")
    sub = lax.axis_index("s")
    _rowcopy(sub, lambda r0, nr: pltpu.sync_copy(
        z_hbm.at[pl.ds(core * N + r0, nr)], acc.at[pl.ds(r0, nr)]))
    _rowcopy(sub, lambda r0, nr: pltpu.sync_copy(
        z_hbm.at[pl.ds(core * N + r0, nr)], z_s.at[pl.ds(r0, nr)]))
    plsc.subcore_barrier()

    base = sub * EPS
    rows = (rows0, rows1)
    semi = (semi0, semi1)
    semg = (semg0, semg1)

    def idx_start(c, p):
        off = base + c * CHUNK
        pltpu.async_copy(src_hbm.at[pl.ds(off, CHUNK)], sb.at[p], semi[p])
        pltpu.async_copy(dst_hbm.at[pl.ds(off, CHUNK)], db.at[p], semi[p])

    def idx_wait(p):
        pltpu.make_async_copy(src_hbm.at[pl.ds(0, CHUNK)], sb.at[p], semi[p]).wait()
        pltpu.make_async_copy(dst_hbm.at[pl.ds(0, CHUNK)], db.at[p], semi[p]).wait()

    def transform(p):
        for j in range(CHUNK // 16):
            sl = pl.ds(j * 16, 16)
            s16 = sb[p, sl]
            d16 = db[p, sl]
            gb[p, sl] = s16
            wb[p, sl] = jnp.where(s16 == d16, GARB, d16)

    def gather_start(p):
        pltpu.async_copy(z_s.at[gb.at[p]], rows[p], semg[p])

    def gather_wait(p):
        pltpu.make_async_copy(z_s.at[gb.at[p]], rows[p], semg[p]).wait()

    def scatter(p):
        pltpu.sync_copy(rows[p], acc.at[wb.at[p]], add=True)

    idx_start(0, 0)
    idx_start(1, 1)
    idx_wait(0)
    transform(0)
    gather_start(0)
    idx_start(2, 0)

    def body(k2, carry):
        idx_wait(1)
        transform(1)
        gather_start(1)

        @pl.when(k2 < NK2 - 1)
        def _():
            idx_start(2 * k2 + 3, 1)

        gather_wait(0)
        scatter(0)

        @pl.when(k2 < NK2 - 1)
        def _():
            idx_wait(0)
            transform(0)
            gather_start(0)

        @pl.when(k2 < NK2 - 2)
        def _():
            idx_start(2 * k2 + 4, 0)

        gather_wait(1)
        scatter(1)
        return carry

    lax.fori_loop(0, NK2, body, 0)
    plsc.subcore_barrier()
    _rowcopy(sub, lambda r0, nr: pltpu.sync_copy(
        acc.at[pl.ds(r0, nr)], out_hbm.at[pl.ds(core * N + r0, nr)]))


@functools.partial(
    pl.kernel,
    out_type=jax.ShapeDtypeStruct((NCORE * N, 16), jnp.float32),
    mesh=_MESH,
    compiler_params=pltpu.CompilerParams(use_tc_tiling_on_sc=False),
    scratch_types=[
        pltpu.VMEM((2, CHUNK), jnp.int32),
        pltpu.VMEM((2, CHUNK), jnp.int32),
        pltpu.VMEM((2, CHUNK), jnp.int32),
        pltpu.VMEM((CHUNK, 16), jnp.float32),
        pltpu.VMEM_SHARED((ACC_ROWS, 16), jnp.float32),
        pltpu.SemaphoreType.DMA,
        pltpu.SemaphoreType.DMA,
    ],
)
def _degrees(src_hbm, dst_hbm, init_hbm, out_hbm,
             sb, db, wb, ones, acc, semi0, semi1):
    core = lax.axis_index("c")
    sub = lax.axis_index("s")
    pltpu.sync_copy(init_hbm.at[pl.ds(0, CHUNK)], ones)
    _rowcopy(sub, lambda r0, nr: pltpu.sync_copy(
        init_hbm.at[pl.ds(r0, nr)], acc.at[pl.ds(r0, nr)]))
    plsc.subcore_barrier()

    base = sub * EPS
    semi = (semi0, semi1)

    def idx_start(c, p):
        off = base + c * CHUNK
        pltpu.async_copy(src_hbm.at[pl.ds(off, CHUNK)], sb.at[p], semi[p])
        pltpu.async_copy(dst_hbm.at[pl.ds(off, CHUNK)], db.at[p], semi[p])

    def idx_wait(p):
        pltpu.make_async_copy(src_hbm.at[pl.ds(0, CHUNK)], sb.at[p], semi[p]).wait()
        pltpu.make_async_copy(dst_hbm.at[pl.ds(0, CHUNK)], db.at[p], semi[p]).wait()

    def step(p):
        for j in range(CHUNK // 16):
            sl = pl.ds(j * 16, 16)
            s16 = sb[p, sl]
            d16 = db[p, sl]
            i16 = jnp.where(core == 0, s16, d16)
            wb[p, sl] = jnp.where(s16 == d16, GARB, i16)
        pltpu.sync_copy(ones, acc.at[wb.at[p]], add=True)

    idx_start(0, 0)
    idx_start(1, 1)

    def body(k2, carry):
        idx_wait(0)

        @pl.when(k2 < NK2 - 1)
        def _():
            idx_start(2 * k2 + 2, 0)

        step(0)
        idx_wait(1)

        @pl.when(k2 < NK2 - 1)
        def _():
            idx_start(2 * k2 + 3, 1)

        step(1)
        return carry

    lax.fori_loop(0, NK2, body, 0)
    plsc.subcore_barrier()
    _rowcopy(sub, lambda r0, nr: pltpu.sync_copy(
        acc.at[pl.ds(r0, nr)], out_hbm.at[pl.ds(core * N + r0, nr)]))


BN = 2000

_DG_SPEC = pl.BlockSpec((2, BN, 16), lambda c, i: (0, i, 0))


def _pre_body(h_ref, dg_ref, w_ref, o_ref):
    ns = lax.rsqrt(dg_ref[0, :, 0:1])
    o_ref[0] = jnp.dot(h_ref[0] * ns, w_ref[...],
                       preferred_element_type=jnp.float32)


def _pre_call(h, dg, W):
    return pl.pallas_call(
        _pre_body,
        grid=(B, N // BN),
        in_specs=[
            pl.BlockSpec((1, BN, F), lambda c, i: (c, i, 0)),
            _DG_SPEC,
            pl.BlockSpec((F, F), lambda c, i: (0, 0)),
        ],
        out_specs=pl.BlockSpec((1, BN, F), lambda c, i: (c, i, 0)),
        out_shape=jax.ShapeDtypeStruct((B, N, F), jnp.float32),
    )(h, dg, W)


def _mid_body(s_ref, zs_ref, dg_ref, b_ref, w_ref, o_ref):
    nd = lax.rsqrt(dg_ref[1, :, 0:1])
    ns = lax.rsqrt(dg_ref[0, :, 0:1])
    y = (s_ref[0] + zs_ref[0]) * nd + b_ref[...]
    y = jnp.where(y > 0, y, 0.01 * y)
    o_ref[0] = jnp.dot(y * ns, w_ref[...], preferred_element_type=jnp.float32)


def _mid_call(s, zs, dg, b, W):
    Dout = W.shape[1]
    return pl.pallas_call(
        _mid_body,
        grid=(B, N // BN),
        in_specs=[
            pl.BlockSpec((1, BN, F), lambda c, i: (c, i, 0)),
            pl.BlockSpec((1, BN, F), lambda c, i: (c, i, 0)),
            _DG_SPEC,
            pl.BlockSpec((1, F), lambda c, i: (0, 0)),
            pl.BlockSpec((F, Dout), lambda c, i: (0, 0)),
        ],
        out_specs=pl.BlockSpec((1, BN, Dout), lambda c, i: (c, i, 0)),
        out_shape=jax.ShapeDtypeStruct((B, N, Dout), jnp.float32),
    )(s, zs, dg, b, W)


def _postpre_body(s3_ref, h_ref, dg_ref, b3_ref, w_ref, out_ref, hn_ref, z1_ref):
    nd = lax.rsqrt(dg_ref[1, :, 0:1])
    ns = lax.rsqrt(dg_ref[0, :, 0:1])
    o = s3_ref[0] * nd + b3_ref[...]
    out_ref[0] = o
    hn = jnp.concatenate([h_ref[0][:, 3:], o[:, :3]], axis=1)
    hn_ref[0] = hn
    z1_ref[0] = jnp.dot(hn * ns, w_ref[...], preferred_element_type=jnp.float32)


def _postpre_call(s3, h, dg, b3p, W1):
    return pl.pallas_call(
        _postpre_body,
        grid=(B, N // BN),
        in_specs=[
            pl.BlockSpec((1, BN, 16), lambda c, i: (c, i, 0)),
            pl.BlockSpec((1, BN, F), lambda c, i: (c, i, 0)),
            _DG_SPEC,
            pl.BlockSpec((1, 16), lambda c, i: (0, 0)),
            pl.BlockSpec((F, F), lambda c, i: (0, 0)),
        ],
        out_specs=[
            pl.BlockSpec((1, BN, 16), lambda c, i: (c, i, 0)),
            pl.BlockSpec((1, BN, F), lambda c, i: (c, i, 0)),
            pl.BlockSpec((1, BN, F), lambda c, i: (c, i, 0)),
        ],
        out_shape=[
            jax.ShapeDtypeStruct((B, N, 16), jnp.float32),
            jax.ShapeDtypeStruct((B, N, F), jnp.float32),
            jax.ShapeDtypeStruct((B, N, F), jnp.float32),
        ],
    )(s3, h, dg, b3p, W1)


def _postfin_body(s3_ref, dg_ref, b3_ref, out_ref):
    nd = lax.rsqrt(dg_ref[1, :, 0:1])
    out_ref[0] = s3_ref[0] * nd + b3_ref[...]


def _postfin_call(s3, dg, b3p):
    return pl.pallas_call(
        _postfin_body,
        grid=(B, N // BN),
        in_specs=[
            pl.BlockSpec((1, BN, 16), lambda c, i: (c, i, 0)),
            _DG_SPEC,
            pl.BlockSpec((1, 16), lambda c, i: (0, 0)),
        ],
        out_specs=pl.BlockSpec((1, BN, 16), lambda c, i: (c, i, 0)),
        out_shape=jax.ShapeDtypeStruct((B, N, 16), jnp.float32),
    )(s3, dg, b3p)


def kernel(edge_index, xx, output_length, W1, b1, W2, b2, W3, b3):
    src = edge_index[0].astype(jnp.int32)
    dst = edge_index[1].astype(jnp.int32)
    padn = E_PAD - E
    srcp = jnp.concatenate([src, jnp.zeros((padn,), jnp.int32)])
    dstp = jnp.concatenate([dst, jnp.zeros((padn,), jnp.int32)])

    init16 = jnp.tile(
        (lax.iota(jnp.int32, 16) == 0).astype(jnp.float32)[None, :], (N, 1))
    zeros64 = jnp.zeros((RPS_A, 64), jnp.float32)

    dg = _degrees(srcp, dstp, init16).reshape(NCORE, N, 16)

    b1r = b1.reshape(1, F)
    b2r = b2.reshape(1, F)
    W3p = jnp.pad(W3, ((0, 0), (0, 13)))
    b3p = jnp.pad(b3, (0, 13)).reshape(1, 16)

    def agg_full(z):
        z64 = z.reshape(2 * NCORE * N, 64)
        s64 = _agg128(z64, srcp, dstp, zeros64)
        return s64.reshape(B, N, F)

    def step_to_s3(z1):
        s1 = agg_full(z1)
        z2 = _mid_call(s1, z1, dg, b1r, W2)
        s2 = agg_full(z2)
        p = _mid_call(s2, z2, dg, b2r, W3p)
        s3 = _agg16(p.reshape(NCORE * N, 16), srcp, dstp)
        return s3.reshape(B, N, 16)

    z1 = _pre_call(xx, dg, W1)
    s3 = step_to_s3(z1)
    out0, _h1, z1b = _postpre_call(s3, xx, dg, b3p, W1)
    s3b = step_to_s3(z1b)
    out1 = _postfin_call(s3b, dg, b3p)

    res = jnp.stack([out0[:, :, :3], out1[:, :, :3]], axis=2)
    res = res * (jnp.asarray(output_length) // 2).astype(res.dtype)
    return res
